# baseline (speedup 1.0000x reference)
import functools

import jax
import jax.numpy as jnp
from jax import lax
from jax.experimental import pallas as pl
from jax.experimental.pallas import tpu as pltpu
from jax.experimental.pallas import tpu_sc as plsc

_N = 10000
_NPAD = 10240
_E = 320000
_G = 64
_H = 64
_KEY = 128
_NW = 32
_CH = 128
_CPT = 81
_RPT = _NPAD // 16


def _make_scatter():
    mesh = plsc.VectorSubcoreMesh(core_axis_name="c", subcore_axis_name="s")

    @functools.partial(
        pl.kernel,
        out_type=jax.ShapeDtypeStruct((2, _NPAD, _H), jnp.float32),
        mesh=mesh,
        compiler_params=pltpu.CompilerParams(use_tc_tiling_on_sc=False),
        scratch_types=[
            pltpu.VMEM((_CPT, _CH), jnp.int32),
            pltpu.VMEM((_CPT, _CH), jnp.int32),
            pltpu.VMEM((_CH, _H), jnp.float32),
            pltpu.VMEM((_CH, _H), jnp.float32),
            pltpu.VMEM((_CH, _H), jnp.float32),
            pltpu.VMEM_SHARED((_NPAD, _H), jnp.float32),
            pltpu.VMEM_SHARED((_NPAD, _H), jnp.float32),
            pltpu.SemaphoreType.DMA,
            pltpu.SemaphoreType.DMA,
        ],
    )
    def scat(h_hbm, src_hbm, dst_hbm, zeros_hbm, out_hbm,
             src_v, dst_v, buf_a, buf_b, bounce, agg, hs, sem_a, sem_b):
        c = lax.axis_index("c")
        s = lax.axis_index("s")
        wid = c * 16 + s

        pltpu.sync_copy(zeros_hbm, bounce)

        @pl.loop(0, _RPT // _CH)
        def _zero(i):
            base = s * _RPT + i * _CH
            pltpu.sync_copy(bounce, agg.at[pl.ds(base, _CH)])

        @pl.loop(0, _RPT // _CH)
        def _stage(i):
            base = s * _RPT + i * _CH
            pltpu.sync_copy(h_hbm.at[pl.ds(base, _CH)], buf_a)
            pltpu.sync_copy(buf_a, hs.at[pl.ds(base, _CH)])

        pltpu.sync_copy(src_hbm.at[wid], src_v)
        pltpu.sync_copy(dst_hbm.at[wid], dst_v)
        plsc.subcore_barrier()

        pltpu.async_copy(hs.at[src_v.at[0]], buf_a, sem_a)

        @pl.loop(0, (_CPT - 1) // 2)
        def _edges(i):
            j = i * 2
            pltpu.async_copy(hs.at[src_v.at[j + 1]], buf_b, sem_b)
            pltpu.make_async_copy(hs.at[src_v.at[j]], buf_a, sem_a).wait()
            pltpu.sync_copy(buf_a, agg.at[dst_v.at[j]], add=True)
            pltpu.async_copy(hs.at[src_v.at[j + 2]], buf_a, sem_a)
            pltpu.make_async_copy(hs.at[src_v.at[j + 1]], buf_b, sem_b).wait()
            pltpu.sync_copy(buf_b, agg.at[dst_v.at[j + 1]], add=True)

        pltpu.make_async_copy(hs.at[src_v.at[_CPT - 1]], buf_a, sem_a).wait()
        pltpu.sync_copy(buf_a, agg.at[dst_v.at[_CPT - 1]], add=True)
        plsc.subcore_barrier()

        @pl.loop(0, _RPT // _CH)
        def _out(i):
            base = s * _RPT + i * _CH
            pltpu.sync_copy(agg.at[pl.ds(base, _CH)], bounce)
            pltpu.sync_copy(bounce, out_hbm.at[c, pl.ds(base, _CH)])

    return scat


_scatter = _make_scatter()

_BLK = 1024
_NBLK = _NPAD // _BLK


def _mlp_pool(z, b3_ref, w1t, w1b, b1_ref, w2_ref, b2_ref, hn_ref, pool_ref,
              zr=None):
    z1 = lax.dot_general(z, w1t, (((1,), (0,)), ((), ())),
                         preferred_element_type=jnp.float32)
    if zr is not None:
        z1 = z1 + lax.dot_general(zr, w1b, (((1,), (0,)), ((), ())),
                                  preferred_element_type=jnp.float32)
    z1 = jnp.maximum(z1 + b1_ref[...], 0.0)
    z2 = lax.dot_general(z1, w2_ref[...], (((1,), (0,)), ((), ())),
                         preferred_element_type=jnp.float32) + b2_ref[...]
    hn = jnp.maximum(z2, 0.0)
    hn_ref[...] = hn
    seg = b3_ref[0, 0, :]
    onehot = (seg[:, None] == lax.broadcasted_iota(jnp.int32, (_BLK, _G), 1)
              ).astype(jnp.float32)
    part = lax.dot_general(onehot, hn, (((0,), (0,)), ((), ())),
                           preferred_element_type=jnp.float32)

    @pl.when(pl.program_id(0) == 0)
    def _():
        pool_ref[...] = part

    @pl.when(pl.program_id(0) > 0)
    def _():
        pool_ref[...] += part


def _layer0_body(aggl_ref, aggr_ref, h_ref, b3_ref, w1_ref, b1_ref, w2_ref,
                 b2_ref, hn_ref, pool_ref):
    zl = aggl_ref[0] + aggl_ref[1] + h_ref[:, 0:_H]
    zr = aggr_ref[0] + aggr_ref[1] + h_ref[:, _H:2 * _H]
    _mlp_pool(zl, b3_ref, w1_ref[0:_H, :], w1_ref[_H:2 * _H, :], b1_ref,
              w2_ref, b2_ref, hn_ref, pool_ref, zr=zr)


def _layer_body(agg_ref, h_ref, b3_ref, w1_ref, b1_ref, w2_ref, b2_ref,
                hn_ref, pool_ref):
    z = agg_ref[0] + agg_ref[1] + h_ref[...]
    _mlp_pool(z, b3_ref, w1_ref[...], None, b1_ref, w2_ref, b2_ref,
              hn_ref, pool_ref)


def _common_specs(din):
    return ([
        pl.BlockSpec((1, 1, _BLK), lambda i: (i, 0, 0)),
        pl.BlockSpec((din, _H), lambda i: (0, 0)),
        pl.BlockSpec((1, _H), lambda i: (0, 0)),
        pl.BlockSpec((_H, _H), lambda i: (0, 0)),
        pl.BlockSpec((1, _H), lambda i: (0, 0)),
    ], [
        pl.BlockSpec((_BLK, _H), lambda i: (i, 0)),
        pl.BlockSpec((_G, _H), lambda i: (0, 0)),
    ], [
        jax.ShapeDtypeStruct((_NPAD, _H), jnp.float32),
        jax.ShapeDtypeStruct((_G, _H), jnp.float32),
    ])


def _make_layer0():
    tail, outs, oshape = _common_specs(2 * _H)
    return pl.pallas_call(
        _layer0_body,
        grid=(_NBLK,),
        in_specs=[
            pl.BlockSpec((2, _BLK, _H), lambda i: (0, i, 0)),
            pl.BlockSpec((2, _BLK, _H), lambda i: (0, i, 0)),
            pl.BlockSpec((_BLK, 2 * _H), lambda i: (i, 0)),
        ] + tail,
        out_specs=outs,
        out_shape=oshape,
    )


def _make_layer():
    tail, outs, oshape = _common_specs(_H)
    return pl.pallas_call(
        _layer_body,
        grid=(_NBLK,),
        in_specs=[
            pl.BlockSpec((2, _BLK, _H), lambda i: (0, i, 0)),
            pl.BlockSpec((_BLK, _H), lambda i: (i, 0)),
        ] + tail,
        out_specs=outs,
        out_shape=oshape,
    )


_layer0 = _make_layer0()
_layer = _make_layer()


def _head_body(emb_ref, w1_ref, b1_ref, w2_ref, b2_ref, out_ref):
    k = lax.dot_general(emb_ref[...], w1_ref[...], (((1,), (0,)), ((), ())),
                        preferred_element_type=jnp.float32) + b1_ref[...]
    k = jnp.where(k > 0, k, 0.01 * k)
    k2 = lax.dot_general(k, w2_ref[...], (((1,), (0,)), ((), ())),
                         preferred_element_type=jnp.float32) + b2_ref[...]
    out_ref[...] = jax.nn.sigmoid(k2)


_head = pl.pallas_call(
    _head_body, out_shape=jax.ShapeDtypeStruct((_G, _KEY), jnp.float32))


def kernel(x, edge_index, batch, params):
    src = edge_index[0].astype(jnp.int32)
    dst = edge_index[1].astype(jnp.int32)
    ept_real = _E // _NW
    npad_edges = _CPT * _CH - ept_real
    pad = _N + ((jnp.arange(npad_edges, dtype=jnp.int32)[None, :]
                 + 7 * jnp.arange(_NW, dtype=jnp.int32)[:, None])
                % (_NPAD - _N))
    srcr = jnp.concatenate([src.reshape(_NW, ept_real), pad],
                           axis=1).reshape(_NW, _CPT, _CH)
    dstr = jnp.concatenate([dst.reshape(_NW, ept_real), pad],
                           axis=1).reshape(_NW, _CPT, _CH)
    batch3 = jnp.concatenate(
        [batch.astype(jnp.int32), jnp.full((_NPAD - _N,), _G, jnp.int32)]
    ).reshape(_NPAD // _BLK, 1, _BLK)
    zeros = jnp.zeros((_CH, _H), jnp.float32)
    h = jnp.pad(x, ((0, _NPAD - _N), (0, 0)))

    (w1, b1, w2, b2) = params["layers"][0]
    aggl = _scatter(h[:, 0:_H], srcr, dstr, zeros)
    aggr = _scatter(h[:, _H:2 * _H], srcr, dstr, zeros)
    h, p = _layer0(aggl, aggr, h, batch3, w1, b1.reshape(1, _H),
                   w2, b2.reshape(1, _H))
    pooled = [p]

    for (w1, b1, w2, b2) in params["layers"][1:]:
        aggp = _scatter(h, srcr, dstr, zeros)
        h, p = _layer(aggp, h, batch3, w1, b1.reshape(1, _H),
                      w2, b2.reshape(1, _H))
        pooled.append(p)

    emb = jnp.concatenate(pooled, axis=1)
    fc1w, fc1b = params["fc1"]
    fc2w, fc2b = params["fc2"]
    return _head(emb, fc1w, fc1b.reshape(1, _H), fc2w, fc2b.reshape(1, _KEY))


# speedup vs baseline: 1.1600x; 1.1600x over previous
import functools

import jax
import jax.numpy as jnp
from jax import lax
from jax.experimental import pallas as pl
from jax.experimental.pallas import tpu as pltpu
from jax.experimental.pallas import tpu_sc as plsc

_N = 10000
_NPAD = 10240
_E = 320000
_G = 64
_H = 64
_KEY = 128
_NW = 32
_CH = 128
_CPT = 81
_RPT = _NPAD // 16


def _make_scatter():
    mesh = plsc.VectorSubcoreMesh(core_axis_name="c", subcore_axis_name="s")

    @functools.partial(
        pl.kernel,
        out_type=jax.ShapeDtypeStruct((2, _NPAD, _H), jnp.float32),
        mesh=mesh,
        compiler_params=pltpu.CompilerParams(use_tc_tiling_on_sc=False),
        scratch_types=[
            pltpu.VMEM((_CPT, _CH), jnp.int32),
            pltpu.VMEM((_CPT, _CH), jnp.int32),
            pltpu.VMEM((_CH, _H), jnp.float32),
            pltpu.VMEM((_CH, _H), jnp.float32),
            pltpu.VMEM((_CH, _H), jnp.float32),
            pltpu.VMEM_SHARED((_NPAD, _H), jnp.float32),
            pltpu.SemaphoreType.DMA,
            pltpu.SemaphoreType.DMA,
        ],
    )
    def scat(h_hbm, src_hbm, dst_hbm, zeros_hbm, out_hbm,
             src_v, dst_v, buf_a, buf_b, bounce, agg, sem_a, sem_b):
        c = lax.axis_index("c")
        s = lax.axis_index("s")
        wid = c * 16 + s

        ci = pltpu.async_copy(src_hbm.at[wid], src_v, sem_a)
        cj = pltpu.async_copy(dst_hbm.at[wid], dst_v, sem_b)

        pltpu.sync_copy(zeros_hbm, bounce)

        @pl.loop(0, _RPT // _CH)
        def _zero(i):
            pltpu.sync_copy(bounce, agg.at[pl.ds(s * _RPT + i * _CH, _CH)])

        ci.wait()
        cj.wait()
        plsc.subcore_barrier()

        pltpu.async_copy(h_hbm.at[src_v.at[0]], buf_a, sem_a)

        @pl.loop(0, (_CPT - 1) // 2)
        def _edges(i):
            j = i * 2
            pltpu.async_copy(h_hbm.at[src_v.at[j + 1]], buf_b, sem_b)
            pltpu.make_async_copy(h_hbm.at[src_v.at[j]], buf_a, sem_a).wait()
            pltpu.sync_copy(buf_a, agg.at[dst_v.at[j]], add=True)
            pltpu.async_copy(h_hbm.at[src_v.at[j + 2]], buf_a, sem_a)
            pltpu.make_async_copy(h_hbm.at[src_v.at[j + 1]], buf_b, sem_b).wait()
            pltpu.sync_copy(buf_b, agg.at[dst_v.at[j + 1]], add=True)

        pltpu.make_async_copy(h_hbm.at[src_v.at[_CPT - 1]], buf_a, sem_a).wait()
        pltpu.sync_copy(buf_a, agg.at[dst_v.at[_CPT - 1]], add=True)
        plsc.subcore_barrier()

        @pl.loop(0, _RPT // _CH)
        def _out(i):
            base = s * _RPT + i * _CH
            pltpu.sync_copy(agg.at[pl.ds(base, _CH)], bounce)
            pltpu.sync_copy(bounce, out_hbm.at[c, pl.ds(base, _CH)])

    return scat


_scatter = _make_scatter()

_BLK = 1024
_NBLK = _NPAD // _BLK


def _mlp_pool(z, b3_ref, w1t, w1b, b1_ref, w2_ref, b2_ref, hn_ref, pool_ref,
              zr=None):
    z1 = lax.dot_general(z, w1t, (((1,), (0,)), ((), ())),
                         preferred_element_type=jnp.float32)
    if zr is not None:
        z1 = z1 + lax.dot_general(zr, w1b, (((1,), (0,)), ((), ())),
                                  preferred_element_type=jnp.float32)
    z1 = jnp.maximum(z1 + b1_ref[...], 0.0)
    z2 = lax.dot_general(z1, w2_ref[...], (((1,), (0,)), ((), ())),
                         preferred_element_type=jnp.float32) + b2_ref[...]
    hn = jnp.maximum(z2, 0.0)
    hn_ref[...] = hn
    seg = b3_ref[0, 0, :]
    onehot = (seg[:, None] == lax.broadcasted_iota(jnp.int32, (_BLK, _G), 1)
              ).astype(jnp.float32)
    part = lax.dot_general(onehot, hn, (((0,), (0,)), ((), ())),
                           preferred_element_type=jnp.float32)

    @pl.when(pl.program_id(0) == 0)
    def _():
        pool_ref[...] = part

    @pl.when(pl.program_id(0) > 0)
    def _():
        pool_ref[...] += part


def _layer0_body(aggl_ref, aggr_ref, h_ref, b3_ref, w1_ref, b1_ref, w2_ref,
                 b2_ref, hn_ref, pool_ref):
    zl = aggl_ref[0] + aggl_ref[1] + h_ref[:, 0:_H]
    zr = aggr_ref[0] + aggr_ref[1] + h_ref[:, _H:2 * _H]
    _mlp_pool(zl, b3_ref, w1_ref[0:_H, :], w1_ref[_H:2 * _H, :], b1_ref,
              w2_ref, b2_ref, hn_ref, pool_ref, zr=zr)


def _layer_body(agg_ref, h_ref, b3_ref, w1_ref, b1_ref, w2_ref, b2_ref,
                hn_ref, pool_ref):
    z = agg_ref[0] + agg_ref[1] + h_ref[...]
    _mlp_pool(z, b3_ref, w1_ref[...], None, b1_ref, w2_ref, b2_ref,
              hn_ref, pool_ref)


def _common_specs(din):
    return ([
        pl.BlockSpec((1, 1, _BLK), lambda i: (i, 0, 0)),
        pl.BlockSpec((din, _H), lambda i: (0, 0)),
        pl.BlockSpec((1, _H), lambda i: (0, 0)),
        pl.BlockSpec((_H, _H), lambda i: (0, 0)),
        pl.BlockSpec((1, _H), lambda i: (0, 0)),
    ], [
        pl.BlockSpec((_BLK, _H), lambda i: (i, 0)),
        pl.BlockSpec((_G, _H), lambda i: (0, 0)),
    ], [
        jax.ShapeDtypeStruct((_NPAD, _H), jnp.float32),
        jax.ShapeDtypeStruct((_G, _H), jnp.float32),
    ])


def _make_layer0():
    tail, outs, oshape = _common_specs(2 * _H)
    return pl.pallas_call(
        _layer0_body,
        grid=(_NBLK,),
        in_specs=[
            pl.BlockSpec((2, _BLK, _H), lambda i: (0, i, 0)),
            pl.BlockSpec((2, _BLK, _H), lambda i: (0, i, 0)),
            pl.BlockSpec((_BLK, 2 * _H), lambda i: (i, 0)),
        ] + tail,
        out_specs=outs,
        out_shape=oshape,
    )


def _make_layer():
    tail, outs, oshape = _common_specs(_H)
    return pl.pallas_call(
        _layer_body,
        grid=(_NBLK,),
        in_specs=[
            pl.BlockSpec((2, _BLK, _H), lambda i: (0, i, 0)),
            pl.BlockSpec((_BLK, _H), lambda i: (i, 0)),
        ] + tail,
        out_specs=outs,
        out_shape=oshape,
    )


_layer0 = _make_layer0()
_layer = _make_layer()


def _head_body(emb_ref, w1_ref, b1_ref, w2_ref, b2_ref, out_ref):
    k = lax.dot_general(emb_ref[...], w1_ref[...], (((1,), (0,)), ((), ())),
                        preferred_element_type=jnp.float32) + b1_ref[...]
    k = jnp.where(k > 0, k, 0.01 * k)
    k2 = lax.dot_general(k, w2_ref[...], (((1,), (0,)), ((), ())),
                         preferred_element_type=jnp.float32) + b2_ref[...]
    out_ref[...] = jax.nn.sigmoid(k2)


_head = pl.pallas_call(
    _head_body, out_shape=jax.ShapeDtypeStruct((_G, _KEY), jnp.float32))


def kernel(x, edge_index, batch, params):
    src = edge_index[0].astype(jnp.int32)
    dst = edge_index[1].astype(jnp.int32)
    ept_real = _E // _NW
    npad_edges = _CPT * _CH - ept_real
    pad = _N + ((jnp.arange(npad_edges, dtype=jnp.int32)[None, :]
                 + 7 * jnp.arange(_NW, dtype=jnp.int32)[:, None])
                % (_NPAD - _N))
    srcr = jnp.concatenate([src.reshape(_NW, ept_real), pad],
                           axis=1).reshape(_NW, _CPT, _CH)
    dstr = jnp.concatenate([dst.reshape(_NW, ept_real), pad],
                           axis=1).reshape(_NW, _CPT, _CH)
    batch3 = jnp.concatenate(
        [batch.astype(jnp.int32), jnp.full((_NPAD - _N,), _G, jnp.int32)]
    ).reshape(_NPAD // _BLK, 1, _BLK)
    zeros = jnp.zeros((_CH, _H), jnp.float32)
    h = jnp.pad(x, ((0, _NPAD - _N), (0, 0)))

    (w1, b1, w2, b2) = params["layers"][0]
    aggl = _scatter(h[:, 0:_H], srcr, dstr, zeros)
    aggr = _scatter(h[:, _H:2 * _H], srcr, dstr, zeros)
    h, p = _layer0(aggl, aggr, h, batch3, w1, b1.reshape(1, _H),
                   w2, b2.reshape(1, _H))
    pooled = [p]

    for (w1, b1, w2, b2) in params["layers"][1:]:
        aggp = _scatter(h, srcr, dstr, zeros)
        h, p = _layer(aggp, h, batch3, w1, b1.reshape(1, _H),
                      w2, b2.reshape(1, _H))
        pooled.append(p)

    emb = jnp.concatenate(pooled, axis=1)
    fc1w, fc1b = params["fc1"]
    fc2w, fc2b = params["fc2"]
    return _head(emb, fc1w, fc1b.reshape(1, _H), fc2w, fc2b.reshape(1, _KEY))
